# baseline (speedup 1.0000x reference)
import functools

import jax
import jax.numpy as jnp
import numpy as np
from jax import lax
from jax.experimental import pallas as pl
from jax.experimental.pallas import tpu as pltpu
from jax.experimental.pallas import tpu_sc as plsc

B = 16384
F = 26
V = 100000
D = 16

NC = 2
NS = 16
L = 16
NW = NC * NS
BPW = B // NW
C = 128
NCHUNK = BPW // C
N = C * F
GW = 128
G = N // GW

_FOFS = np.tile(np.arange(F, dtype=np.int32) * V, C)


def _fm_body(xi_hbm, xv_hbm, w1_hbm, w2_hbm, fofs_hbm, out_hbm,
             idxr_v, idxf_v, fofs_v, xv_v, w1_v, rows_v, ob_v, sem):
    wid = lax.axis_index("s") * NC + lax.axis_index("c")
    base = wid * BPW
    lanes = lax.iota(jnp.int32, L)
    _dn = lax.GatherDimensionNumbers(
        offset_dims=(), collapsed_slice_dims=(0,), start_index_map=(0,))

    def _shuf(x, perm):
        return lax.gather(x, perm[:, None], dimension_numbers=_dn,
                          slice_sizes=(1,),
                          mode=lax.GatherScatterMode.PROMISE_IN_BOUNDS)

    def _lane_sum(x):
        for k in (8, 4, 2, 1):
            x = x + _shuf(x, lanes ^ k)
        return x

    pltpu.sync_copy(fofs_hbm, fofs_v)

    def chunk_body(k, _):
        b0 = pl.multiple_of(base + k * C, 128)
        q0 = pl.multiple_of((base + k * C) * F, 128)

        pltpu.sync_copy(xi_hbm.at[pl.ds(q0, N)], idxr_v)
        pltpu.sync_copy(xv_hbm.at[pl.ds(q0, N)], xv_v.at[pl.ds(0, N)])

        for v in range(N // L):
            vec = idxr_v[pl.ds(v * L, L)] + fofs_v[pl.ds(v * L, L)]
            idxf_v[v * L // GW, pl.ds((v * L) % GW, L)] = vec

        copies = []
        for g in range(G):
            copies.append(pltpu.async_copy(
                w2_hbm.at[idxf_v.at[g]],
                rows_v.at[pl.ds(g * GW, GW)], sem))
            copies.append(pltpu.async_copy(
                w1_hbm.at[idxf_v.at[g]],
                w1_v.at[pl.ds(g * GW, GW)], sem))
        for cp in copies:
            cp.wait()

        def bbody(bg, _):
            ovec = jnp.zeros((L,), jnp.float32)
            for i in range(L):
                jb = (bg * L + i) * F
                xva = xv_v[pl.ds(jb, L)]
                xvb = xv_v[pl.ds(jb + L, L)]
                w1a = w1_v[pl.ds(jb, L)]
                w1b = w1_v[pl.ds(jb + L, L)]
                acc = jnp.zeros((L,), jnp.float32)
                acc2 = jnp.zeros((L,), jnp.float32)
                for f in range(F):
                    xs = xva[f] if f < L else xvb[f - L]
                    t = rows_v[jb + f, :] * xs
                    acc = acc + t
                    acc2 = acc2 + t * t
                total = (jnp.float32(0.5) * (acc * acc - acc2)
                         + xva * w1a
                         + jnp.where(lanes < F - L, xvb * w1b,
                                     jnp.float32(0.0)))
                s = _lane_sum(total)
                ovec = jnp.where(lanes == i, ovec + s, ovec)
            ob_v[pl.ds(bg * L, L)] = ovec
            return 0

        lax.fori_loop(0, C // L, bbody, 0)
        pltpu.sync_copy(ob_v, out_hbm.at[pl.ds(b0, C)])
        return 0

    lax.fori_loop(0, NCHUNK, chunk_body, 0)


@jax.jit
def _fm_kernel(xi_flat, xv_flat, w1_flat, w2_2d):
    fofs = jnp.asarray(_FOFS)
    mesh = plsc.VectorSubcoreMesh(core_axis_name="c", subcore_axis_name="s")
    run = pl.kernel(
        _fm_body,
        mesh=mesh,
        compiler_params=pltpu.CompilerParams(use_tc_tiling_on_sc=False),
        out_type=jax.ShapeDtypeStruct((B,), jnp.float32),
        scratch_types=[
            pltpu.VMEM((N,), jnp.int32),
            pltpu.VMEM((G, GW), jnp.int32),
            pltpu.VMEM((N,), jnp.int32),
            pltpu.VMEM((N + L,), jnp.float32),
            pltpu.VMEM((N + L,), jnp.float32),
            pltpu.VMEM((N, D), jnp.float32),
            pltpu.VMEM((C,), jnp.float32),
            pltpu.SemaphoreType.DMA,
        ],
    )
    return run(xi_flat, xv_flat, w1_flat, w2_2d, fofs)


def kernel(Xi, Xv, W1, W2, bias):
    xi_flat = Xi.reshape(B * F).astype(jnp.int32)
    xv_flat = Xv.reshape(B * F)
    w1_flat = W1.reshape(F * V)
    w2_2d = W2.reshape(F * V, D)
    return _fm_kernel(xi_flat, xv_flat, w1_flat, w2_2d) + bias


# speedup vs baseline: 1.0016x; 1.0016x over previous
import functools

import jax
import jax.numpy as jnp
import numpy as np
from jax import lax
from jax.experimental import pallas as pl
from jax.experimental.pallas import tpu as pltpu
from jax.experimental.pallas import tpu_sc as plsc

B = 16384
F = 26
V = 100000
D = 16

NC = 2
NS = 16
L = 16
NW = NC * NS
BPW = B // NW
C = 128
NCHUNK = BPW // C
N = C * F
NFB = (F + L - 1) // L


def _fm_body(xi_hbm, xv_hbm, w1_hbm, w2_hbm, out_hbm,
             idxr_v, idxt_v, idxw_v, xv_v, xvt_v, w1t_v, rows_v, ob_v, sem):
    wid = lax.axis_index("s") * NC + lax.axis_index("c")
    base = wid * BPW
    lanes = lax.iota(jnp.int32, L)
    _dn = lax.GatherDimensionNumbers(
        offset_dims=(), collapsed_slice_dims=(0,), start_index_map=(0,))

    def _shuf(x, perm):
        return lax.gather(x, perm[:, None], dimension_numbers=_dn,
                          slice_sizes=(1,),
                          mode=lax.GatherScatterMode.PROMISE_IN_BOUNDS)

    def _lane_sum(x):
        for k in (8, 4, 2, 1):
            x = x + _shuf(x, lanes ^ k)
        return x

    def _transpose16(vs):
        for k in (8, 4, 2, 1):
            vs = [jnp.where((lanes & k) == (i & k), vs[i],
                            _shuf(vs[i ^ k], lanes ^ k))
                  for i in range(L)]
        return vs

    def chunk_body(ck, _):
        b0 = pl.multiple_of(base + ck * C, 128)
        q0 = pl.multiple_of((base + ck * C) * F, 128)

        pltpu.sync_copy(xi_hbm.at[pl.ds(q0, N)], idxr_v.at[pl.ds(0, N)])
        pltpu.sync_copy(xv_hbm.at[pl.ds(q0, N)], xv_v.at[pl.ds(0, N)])

        def tbody(bg, _):
            jb = bg * (L * F)
            for fb in range(NFB):
                nf = min(L, F - fb * L)
                iv = [idxr_v[pl.ds(jb + i * F + fb * L, L)] for i in range(L)]
                vv = [xv_v[pl.ds(jb + i * F + fb * L, L)] for i in range(L)]
                it = _transpose16(iv)
                vt = _transpose16(vv)
                for j in range(nf):
                    f = fb * L + j
                    idxt_v[f, pl.ds(bg * L, L)] = it[j]
                    idxw_v[f, pl.ds(bg * L, L)] = it[j] + jnp.int32(f * V)
                    xvt_v[f, pl.ds(bg * L, L)] = vt[j]
            return 0

        lax.fori_loop(0, C // L, tbody, 0)

        copies = []
        for f in range(F):
            copies.append(pltpu.async_copy(
                w2_hbm.at[f].at[idxt_v.at[f]],
                rows_v.at[pl.ds(f * C, C), :], sem))
            copies.append(pltpu.async_copy(
                w1_hbm.at[idxw_v.at[f]],
                w1t_v.at[pl.ds(f * C, C)], sem))
        for cp in copies:
            cp.wait()

        def bbody(bg, _):
            facc = jnp.zeros((L,), jnp.float32)
            xvl = []
            for f in range(F):
                xvrow = xvt_v[f, pl.ds(bg * L, L)]
                w1row = w1t_v[pl.ds(f * C + bg * L, L)]
                facc = facc + w1row * xvrow
                xvl.append(xvrow)
            ovec = facc
            for i in range(L):
                b = bg * L + i
                acc = jnp.zeros((L,), jnp.float32)
                acc2 = jnp.zeros((L,), jnp.float32)
                for f in range(F):
                    t = rows_v[f * C + b, :] * xvl[f][i]
                    acc = acc + t
                    acc2 = acc2 + t * t
                s = _lane_sum(acc * acc - acc2)
                ovec = jnp.where(lanes == i, ovec + jnp.float32(0.5) * s, ovec)
            ob_v[pl.ds(bg * L, L)] = ovec
            return 0

        lax.fori_loop(0, C // L, bbody, 0)
        pltpu.sync_copy(ob_v, out_hbm.at[pl.ds(b0, C)])
        return 0

    lax.fori_loop(0, NCHUNK, chunk_body, 0)


@jax.jit
def _fm_kernel(xi_flat, xv_flat, w1_flat, w2_3d):
    mesh = plsc.VectorSubcoreMesh(core_axis_name="c", subcore_axis_name="s")
    run = pl.kernel(
        _fm_body,
        mesh=mesh,
        compiler_params=pltpu.CompilerParams(use_tc_tiling_on_sc=False),
        out_type=jax.ShapeDtypeStruct((B,), jnp.float32),
        scratch_types=[
            pltpu.VMEM((N + L,), jnp.int32),
            pltpu.VMEM((F, C), jnp.int32),
            pltpu.VMEM((F, C), jnp.int32),
            pltpu.VMEM((N + L,), jnp.float32),
            pltpu.VMEM((F, C), jnp.float32),
            pltpu.VMEM((N,), jnp.float32),
            pltpu.VMEM((N, D), jnp.float32),
            pltpu.VMEM((C,), jnp.float32),
            pltpu.SemaphoreType.DMA,
        ],
    )
    return run(xi_flat, xv_flat, w1_flat, w2_3d)


def kernel(Xi, Xv, W1, W2, bias):
    xi_flat = Xi.reshape(B * F).astype(jnp.int32)
    xv_flat = Xv.reshape(B * F)
    w1_flat = W1.reshape(F * V)
    return _fm_kernel(xi_flat, xv_flat, w1_flat, W2) + bias


# speedup vs baseline: 1.0021x; 1.0005x over previous
import functools

import jax
import jax.numpy as jnp
import numpy as np
from jax import lax
from jax.experimental import pallas as pl
from jax.experimental.pallas import tpu as pltpu
from jax.experimental.pallas import tpu_sc as plsc

B = 16384
F = 26
V = 100000
D = 16

NC = 2
NS = 16
L = 16
NW = NC * NS
BPW = B // NW
C = 128
NCHUNK = BPW // C
N = C * F
NFB = (F + L - 1) // L


def _fm_body(xi_hbm, xv_hbm, w1_hbm, w2_hbm, out_hbm,
             idxr_v, idxt_v, xv_v, xvt_v, w1t_v, rows_v, ob_v, sem):
    wid = lax.axis_index("s") * NC + lax.axis_index("c")
    base = wid * BPW
    lanes = lax.iota(jnp.int32, L)
    _dn = lax.GatherDimensionNumbers(
        offset_dims=(), collapsed_slice_dims=(0,), start_index_map=(0,))

    def _shuf(x, perm):
        return lax.gather(x, perm[:, None], dimension_numbers=_dn,
                          slice_sizes=(1,),
                          mode=lax.GatherScatterMode.PROMISE_IN_BOUNDS)

    def _lane_sum(x):
        for k in (8, 4, 2, 1):
            x = x + _shuf(x, lanes ^ k)
        return x

    def _transpose16(vs):
        for k in (8, 4, 2, 1):
            vs = [jnp.where((lanes & k) == (i & k), vs[i],
                            _shuf(vs[i ^ k], lanes ^ k))
                  for i in range(L)]
        return vs

    def chunk_body(ck, _):
        b0 = pl.multiple_of(base + ck * C, 128)
        q0 = pl.multiple_of((base + ck * C) * F, 128)

        pltpu.sync_copy(xi_hbm.at[pl.ds(q0, N)], idxr_v.at[pl.ds(0, N)])
        pltpu.sync_copy(xv_hbm.at[pl.ds(q0, N)], xv_v.at[pl.ds(0, N)])

        def tbody(bg, _):
            jb = bg * (L * F)
            for fb in range(NFB):
                nf = min(L, F - fb * L)
                iv = [idxr_v[pl.ds(jb + i * F + fb * L, L)] for i in range(L)]
                vv = [xv_v[pl.ds(jb + i * F + fb * L, L)] for i in range(L)]
                it = _transpose16(iv)
                vt = _transpose16(vv)
                for j in range(nf):
                    f = fb * L + j
                    idxt_v[f, pl.ds(bg * L, L)] = it[j]
                    xvt_v[f, pl.ds(bg * L, L)] = vt[j]
            return 0

        lax.fori_loop(0, C // L, tbody, 0)

        copies = []
        for f in range(F):
            copies.append(pltpu.async_copy(
                w2_hbm.at[f].at[idxt_v.at[f]],
                rows_v.at[pl.ds(f * C, C), :], sem))
            copies.append(pltpu.async_copy(
                w1_hbm.at[f].at[idxt_v.at[f]],
                w1t_v.at[pl.ds(f * C, C)], sem))
        for cp in copies:
            cp.wait()

        def bbody(bg, _):
            facc = jnp.zeros((L,), jnp.float32)
            xvl = []
            for f in range(F):
                xvrow = xvt_v[f, pl.ds(bg * L, L)]
                w1row = w1t_v[pl.ds(f * C + bg * L, L)]
                facc = facc + w1row * xvrow
                xvl.append(xvrow)
            ovec = facc
            for i in range(L):
                b = bg * L + i
                acc = jnp.zeros((L,), jnp.float32)
                acc2 = jnp.zeros((L,), jnp.float32)
                for f in range(F):
                    t = rows_v[f * C + b, :] * xvl[f][i]
                    acc = acc + t
                    acc2 = acc2 + t * t
                s = _lane_sum(acc * acc - acc2)
                ovec = jnp.where(lanes == i, ovec + jnp.float32(0.5) * s, ovec)
            ob_v[pl.ds(bg * L, L)] = ovec
            return 0

        lax.fori_loop(0, C // L, bbody, 0)
        pltpu.sync_copy(ob_v, out_hbm.at[pl.ds(b0, C)])
        return 0

    lax.fori_loop(0, NCHUNK, chunk_body, 0)


@jax.jit
def _fm_kernel(xi_flat, xv_flat, w1_flat, w2_3d):
    mesh = plsc.VectorSubcoreMesh(core_axis_name="c", subcore_axis_name="s")
    run = pl.kernel(
        _fm_body,
        mesh=mesh,
        compiler_params=pltpu.CompilerParams(use_tc_tiling_on_sc=False),
        out_type=jax.ShapeDtypeStruct((B,), jnp.float32),
        scratch_types=[
            pltpu.VMEM((N + L,), jnp.int32),
            pltpu.VMEM((F, C), jnp.int32),
            pltpu.VMEM((N + L,), jnp.float32),
            pltpu.VMEM((F, C), jnp.float32),
            pltpu.VMEM((N,), jnp.float32),
            pltpu.VMEM((N, D), jnp.float32),
            pltpu.VMEM((C,), jnp.float32),
            pltpu.SemaphoreType.DMA,
        ],
    )
    return run(xi_flat, xv_flat, w1_flat, w2_3d)


def kernel(Xi, Xv, W1, W2, bias):
    xi_flat = Xi.reshape(B * F).astype(jnp.int32)
    xv_flat = Xv.reshape(B * F)
    w1_2d = W1.transpose(0, 2, 1).reshape(F, V)
    return _fm_kernel(xi_flat, xv_flat, w1_2d, W2) + bias
